# baseline (speedup 1.0000x reference)
import functools

import jax
import jax.numpy as jnp
from jax import lax
from jax.experimental import pallas as pl
from jax.experimental.pallas import tpu as pltpu
from jax.experimental.pallas import tpu_sc as plsc

B = 16384
HID = 64
NC = 2
NS = 16
NW = NC * NS
BPW = B // NW
ICH = 128
NCH = BPW // ICH
G = 16
NG = BPW // G


def _score_body(h_idx, r_idx, t_idx, ent, rel, out,
                hi_v, ri_v, ti_v, h_v, r_v, t_v, tmp_v, s_v, sem):
    wid = lax.axis_index("s") * NC + lax.axis_index("c")
    base = wid * BPW

    pltpu.sync_copy(h_idx.at[pl.ds(wid * NCH, NCH)], hi_v)
    pltpu.sync_copy(r_idx.at[pl.ds(wid * NCH, NCH)], ri_v)
    pltpu.sync_copy(t_idx.at[pl.ds(wid * NCH, NCH)], ti_v)

    copies = []
    for j in range(NCH):
        dst = pl.ds(j * ICH, ICH)
        copies.append(pltpu.async_copy(ent.at[hi_v.at[j]], h_v.at[dst], sem))
        copies.append(pltpu.async_copy(rel.at[ri_v.at[j]], r_v.at[dst], sem))
        copies.append(pltpu.async_copy(ent.at[ti_v.at[j]], t_v.at[dst], sem))
    for c in copies:
        c.wait()

    lane16 = lax.iota(jnp.int32, 16) * 16

    def group(g, carry):
        for j in range(G):
            row = g * G + j
            p = jnp.zeros((16,), jnp.float32)
            for c in range(HID // 16):
                d = pl.ds(c * 16, 16)
                p = p + h_v[row, d] * r_v[row, d] * t_v[row, d]
            tmp_v[pl.ds(j * 16, 16)] = p
        acc = jnp.zeros((16,), jnp.float32)
        for l in range(16):
            acc = acc + plsc.load_gather(tmp_v, [lane16 + l])
        s_v[pl.ds(g * G, G)] = acc
        return carry

    lax.fori_loop(0, NG, group, 0)

    pltpu.sync_copy(s_v, out.at[pl.ds(base, BPW)])


@jax.jit
def _scores(h_idx, r_idx, t_idx, ent_emb, rel_emb):
    mesh = plsc.VectorSubcoreMesh(core_axis_name="c", subcore_axis_name="s")
    return pl.kernel(
        _score_body,
        mesh=mesh,
        compiler_params=pltpu.CompilerParams(
            needs_layout_passes=False, use_tc_tiling_on_sc=False),
        out_type=jax.ShapeDtypeStruct((B,), jnp.float32),
        scratch_types=[
            pltpu.VMEM((NCH, ICH), jnp.int32),
            pltpu.VMEM((NCH, ICH), jnp.int32),
            pltpu.VMEM((NCH, ICH), jnp.int32),
            pltpu.VMEM((BPW, HID), jnp.float32),
            pltpu.VMEM((BPW, HID), jnp.float32),
            pltpu.VMEM((BPW, HID), jnp.float32),
            pltpu.VMEM((G * 16,), jnp.float32),
            pltpu.VMEM((BPW,), jnp.float32),
            pltpu.SemaphoreType.DMA,
        ],
    )(h_idx, r_idx, t_idx, ent_emb, rel_emb)


def kernel(triples, ent_emb, rel_emb):
    h_idx = triples[:, 0].astype(jnp.int32).reshape(NW * NCH, ICH)
    r_idx = triples[:, 1].astype(jnp.int32).reshape(NW * NCH, ICH)
    t_idx = triples[:, 2].astype(jnp.int32).reshape(NW * NCH, ICH)
    scores = _scores(h_idx, r_idx, t_idx, ent_emb, rel_emb)
    return (scores, jnp.zeros(()))


# speedup vs baseline: 1.1024x; 1.1024x over previous
import functools

import jax
import jax.numpy as jnp
from jax import lax
from jax.experimental import pallas as pl
from jax.experimental.pallas import tpu as pltpu
from jax.experimental.pallas import tpu_sc as plsc

B = 16384
HID = 64
PAD = 128
NC = 2
NS = 16
NW = NC * NS
BPW = B // NW
ICH = 128
NCH = BPW // ICH
RND = 2
CPR = NCH // RND
TPR = BPW // RND
G = 16
NG = TPR // G


def _score_body(h_idx, r_idx, t_idx, ent, rel, out,
                hi_v, ri_v, ti_v, h_v, r_v, t_v, tmp_v, s_v, sem):
    wid = lax.axis_index("s") * NC + lax.axis_index("c")

    pltpu.sync_copy(h_idx.at[wid], hi_v)
    pltpu.sync_copy(r_idx.at[wid], ri_v)
    pltpu.sync_copy(t_idx.at[wid], ti_v)

    lane16 = lax.iota(jnp.int32, 16) * 16

    for rnd in range(RND):
        copies = []
        for j in range(CPR):
            dst = pl.ds(j * ICH, ICH)
            ji = rnd * CPR + j
            copies.append(pltpu.async_copy(ent.at[hi_v.at[ji]], h_v.at[dst], sem))
            copies.append(pltpu.async_copy(rel.at[ri_v.at[ji]], r_v.at[dst], sem))
            copies.append(pltpu.async_copy(ent.at[ti_v.at[ji]], t_v.at[dst], sem))
        for c in copies:
            c.wait()

        def group(g, carry):
            for j in range(G):
                row = g * G + j
                p = jnp.zeros((16,), jnp.float32)
                for c in range(HID // 16):
                    d = pl.ds(c * 16, 16)
                    p = p + h_v[row, d] * r_v[row, d] * t_v[row, d]
                tmp_v[pl.ds(j * 16, 16)] = p
            acc = jnp.zeros((16,), jnp.float32)
            for l in range(16):
                acc = acc + plsc.load_gather(tmp_v, [lane16 + l])
            s_v[pl.ds(rnd * TPR + g * G, G)] = acc
            return carry

        lax.fori_loop(0, NG, group, 0)

    pltpu.sync_copy(s_v, out.at[pl.ds(wid * BPW, BPW)])


@jax.jit
def _scores(h_idx, r_idx, t_idx, ent_emb, rel_emb):
    mesh = plsc.VectorSubcoreMesh(core_axis_name="c", subcore_axis_name="s")
    return pl.kernel(
        _score_body,
        mesh=mesh,
        compiler_params=pltpu.CompilerParams(
            needs_layout_passes=False, use_tc_tiling_on_sc=True),
        out_type=jax.ShapeDtypeStruct((B,), jnp.float32),
        scratch_types=[
            pltpu.VMEM((NCH, ICH), jnp.int32),
            pltpu.VMEM((NCH, ICH), jnp.int32),
            pltpu.VMEM((NCH, ICH), jnp.int32),
            pltpu.VMEM((TPR, PAD), jnp.float32),
            pltpu.VMEM((TPR, PAD), jnp.float32),
            pltpu.VMEM((TPR, PAD), jnp.float32),
            pltpu.VMEM((G * 16,), jnp.float32),
            pltpu.VMEM((BPW,), jnp.float32),
            pltpu.SemaphoreType.DMA,
        ],
    )(h_idx, r_idx, t_idx, ent_emb, rel_emb)


def kernel(triples, ent_emb, rel_emb):
    ent_p = jnp.pad(ent_emb, ((0, 0), (0, PAD - HID)))
    rel_p = jnp.pad(rel_emb, ((0, 0), (0, PAD - HID)))
    h_idx = triples[:, 0].astype(jnp.int32).reshape(NW, NCH, ICH)
    r_idx = triples[:, 1].astype(jnp.int32).reshape(NW, NCH, ICH)
    t_idx = triples[:, 2].astype(jnp.int32).reshape(NW, NCH, ICH)
    scores = _scores(h_idx, r_idx, t_idx, ent_p, rel_p)
    return (scores, jnp.zeros(()))
